# baseline (speedup 1.0000x reference)
import functools

import jax
import jax.numpy as jnp
from jax import lax
from jax.experimental import pallas as pl
from jax.experimental.pallas import tpu as pltpu
from jax.experimental.pallas import tpu_sc as plsc

_N = 8192
_FEAT = 64
_HID = 64
_MAP = 8
_NETS = _MAP * _MAP
_FREQS = 4
_IN1 = _FEAT + 3 + 3 * 2 * _FREQS
_IN1P = 96
_TILE = 128
_HALF = 12288
_SLOTS = 2 * _HALF
_TILES = _SLOTS // _TILE
_TPW = 6
_SPW = _TPW * _TILE


def _route_body(nx_hbm, ny_hbm, feat_hbm, view_hbm,
                perm_hbm, tid_hbm, xsf_hbm, xsv_hbm,
                nx_v, ny_v, list0, list1, sent_v, rowbuf, cnt_rd,
                pc_v, pcc_v, featrows, viewrows, tid_row, cnt_sh, sem):
    c = lax.axis_index("c")
    s = lax.axis_index("s")
    half = c * _HALF
    lanes = lax.iota(jnp.int32, 16)
    eglob0 = 32 * c + 2 * s
    eglob1 = eglob0 + 1

    pltpu.sync_copy(nx_hbm, nx_v)
    pltpu.sync_copy(ny_hbm, ny_v)

    def step(i, carry):
        cur0, cur1 = carry
        vx = nx_v[pl.ds(i * 16, 16)]
        vy = ny_v[pl.ds(i * 16, 16)]
        cxi = jnp.clip(vx * float(_MAP), 0.0, _MAP - 1.0)
        cyi = jnp.clip(vy * float(_MAP), 0.0, _MAP - 1.0)
        eid = cxi.astype(jnp.int32) * _MAP + cyi.astype(jnp.int32)
        pid = i * 16 + lanes
        m0 = eid == eglob0
        mi0 = m0.astype(jnp.int32)
        r0 = plsc.cumsum(mi0) - mi0
        plsc.store_scatter(list0, [cur0 + r0], pid, mask=m0)
        cur0 = cur0 + plsc.all_reduce_population_count(m0)
        m1 = eid == eglob1
        mi1 = m1.astype(jnp.int32)
        r1 = plsc.cumsum(mi1) - mi1
        plsc.store_scatter(list1, [cur1 + r1], pid, mask=m1)
        cur1 = cur1 + plsc.all_reduce_population_count(m1)
        return cur0, cur1

    zero = jnp.zeros((16,), jnp.int32)
    cur0, cur1 = lax.fori_loop(0, _N // 16, step, (zero, zero))

    rowbuf[...] = jnp.where(lanes == 0, cur0, jnp.where(lanes == 1, cur1, 0))
    pltpu.sync_copy(rowbuf, cnt_sh.at[s])
    plsc.subcore_barrier()
    pltpu.sync_copy(cnt_sh, cnt_rd)

    k0 = lanes
    k1 = lanes + 16
    c0v = plsc.load_gather(cnt_rd, [k0 // 2, k0 % 2])
    c1v = plsc.load_gather(cnt_rd, [k1 // 2, k1 % 2])
    p0v = ((c0v + (_TILE - 1)) // _TILE) * _TILE
    p1v = ((c1v + (_TILE - 1)) // _TILE) * _TILE
    s0 = plsc.cumsum(p0v)
    off0 = s0 - p0v
    tot0 = jnp.max(s0)
    s1 = plsc.cumsum(p1v)
    off1 = s1 - p1v + tot0
    gt = tot0 + jnp.max(s1)

    def sel(vec0, vec1, k):
        v = jnp.where(k < 16, vec0, vec1)
        l = jnp.where(k < 16, k, k - 16)
        return jnp.sum(jnp.where(lanes == l, v, 0))

    dst0 = sel(off0, off1, 2 * s)
    dst1 = sel(off0, off1, 2 * s + 1)
    c0 = jnp.max(cur0)
    c1 = jnp.max(cur1)

    sent = jnp.full((16,), _N, jnp.int32)
    cb0 = pl.multiple_of((c0 // 16) * 16, 16)
    cb1 = pl.multiple_of((c1 // 16) * 16, 16)
    for m in range(9):
        pos0 = cb0 + 16 * m + lanes
        v0 = list0[pl.ds(cb0 + 16 * m, 16)]
        list0[pl.ds(cb0 + 16 * m, 16)] = jnp.where(pos0 >= c0, sent, v0)
        pos1 = cb1 + 16 * m + lanes
        v1 = list1[pl.ds(cb1 + 16 * m, 16)]
        list1[pl.ds(cb1 + 16 * m, 16)] = jnp.where(pos1 >= c1, sent, v1)
    for m in range(8):
        sent_v[pl.ds(16 * m, 16)] = sent

    nch0 = (c0 + (_TILE - 1)) // _TILE
    nch1 = (c1 + (_TILE - 1)) // _TILE

    def copy_chunks(j, carry):
        jt = pl.multiple_of(j * _TILE, _TILE)

        @pl.when(j < nch0)
        def _():
            pltpu.sync_copy(
                list0.at[pl.ds(jt, _TILE)],
                perm_hbm.at[pl.ds(pl.multiple_of(half + dst0 + j * _TILE,
                                                 _TILE), _TILE)])

        @pl.when(j < nch1)
        def _():
            pltpu.sync_copy(
                list1.at[pl.ds(jt, _TILE)],
                perm_hbm.at[pl.ds(pl.multiple_of(half + dst1 + j * _TILE,
                                                 _TILE), _TILE)])

        return carry

    lax.fori_loop(0, _N // _TILE, copy_chunks, 0)

    gtc = gt // _TILE
    for m in range(_TPW):
        j = s + 16 * m

        @pl.when(j >= gtc)
        def _():
            pltpu.sync_copy(
                sent_v,
                perm_hbm.at[pl.ds(pl.multiple_of(half + j * _TILE, _TILE),
                                  _TILE)])

    plsc.subcore_barrier()

    base = pl.multiple_of(half + _SPW * s, _TILE)
    for m in range(_TPW):
        pltpu.sync_copy(perm_hbm.at[pl.ds(base + _TILE * m, _TILE)], pc_v.at[m])
    for m in range(_TPW):
        for q in range(_TILE // 16):
            v = pc_v[m, pl.ds(16 * q, 16)]
            pcc_v[m, pl.ds(16 * q, 16)] = jnp.minimum(v, _N - 1)

    for m in range(_TPW):
        pltpu.async_copy(feat_hbm.at[pcc_v.at[m]],
                         featrows.at[pl.ds(m * _TILE, _TILE)], sem).wait()
        pltpu.async_copy(view_hbm.at[pcc_v.at[m]],
                         viewrows.at[pl.ds(m * _TILE, _TILE)], sem).wait()
    pltpu.sync_copy(featrows, xsf_hbm.at[pl.ds(base, _SPW)])
    pltpu.sync_copy(viewrows, xsv_hbm.at[pl.ds(base, _SPW)])

    trow = jnp.zeros((16,), jnp.int32)
    for m in range(_TPW):
        v = pcc_v[m, pl.ds(0, 16)]
        p = jnp.sum(jnp.where(lanes == 0, v, 0))
        pvec = jnp.zeros((16,), jnp.int32) + p
        gx = plsc.load_gather(nx_v, [pvec])
        gy = plsc.load_gather(ny_v, [pvec])
        ex = jnp.clip(gx * float(_MAP), 0.0, _MAP - 1.0)
        ey = jnp.clip(gy * float(_MAP), 0.0, _MAP - 1.0)
        em = ex.astype(jnp.int32) * _MAP + ey.astype(jnp.int32)
        trow = jnp.where(lanes == m, em, trow)
    tid_row[...] = trow
    pltpu.sync_copy(tid_row, tid_hbm.at[16 * c + s])


def _mlp_body(eid_ref, xf_ref, xv_ref, w1_ref, b1_ref, w2_ref, ys_ref):
    t = pl.program_id(0)
    e = eid_ref[t]
    xf = xf_ref[...]
    xv = xv_ref[...]
    v = xv[:, :3]
    ang = jnp.concatenate([v * ((2.0 ** k) * jnp.pi) for k in range(_FREQS)],
                          axis=1)
    x = jnp.concatenate(
        [xf, v, jnp.sin(ang), jnp.cos(ang),
         jnp.zeros((xf.shape[0], _IN1P - _IN1), xf.dtype)], axis=1)
    h = jax.lax.dot_general(x, w1_ref[e], (((1,), (0,)), ((), ())),
                            preferred_element_type=jnp.float32)
    h = jax.nn.relu(h + b1_ref[e])
    ys_ref[...] = jax.lax.dot_general(h, w2_ref[e], (((1,), (0,)), ((), ())),
                                      preferred_element_type=jnp.float32)


def _scatter_body(perm_hbm, ys_hbm, outp_hbm, pc_v, rows_v, sem):
    c = lax.axis_index("c")
    s = lax.axis_index("s")
    base = pl.multiple_of((16 * c + s) * _SPW, _TILE)
    for m in range(_TPW):
        pltpu.sync_copy(perm_hbm.at[pl.ds(base + _TILE * m, _TILE)], pc_v.at[m])
    pltpu.sync_copy(ys_hbm.at[pl.ds(base, _SPW)], rows_v)
    for m in range(_TPW):
        pltpu.async_copy(rows_v.at[pl.ds(m * _TILE, _TILE)],
                         outp_hbm.at[pc_v.at[m]], sem).wait()


def kernel(norm, feat, viewdir, kn_params):
    mesh = plsc.VectorSubcoreMesh(core_axis_name="c", subcore_axis_name="s")
    sc_params = pltpu.CompilerParams(needs_layout_passes=False,
                                     use_tc_tiling_on_sc=False)

    route = pl.kernel(
        _route_body,
        out_type=[
            jax.ShapeDtypeStruct((_SLOTS,), jnp.int32),
            jax.ShapeDtypeStruct((32, 16), jnp.int32),
            jax.ShapeDtypeStruct((_SLOTS, _FEAT), jnp.float32),
            jax.ShapeDtypeStruct((_SLOTS, 16), jnp.float32),
        ],
        mesh=mesh,
        scratch_types=[
            pltpu.VMEM((_N,), jnp.float32),
            pltpu.VMEM((_N,), jnp.float32),
            pltpu.VMEM((_N + _TILE,), jnp.int32),
            pltpu.VMEM((_N + _TILE,), jnp.int32),
            pltpu.VMEM((_TILE,), jnp.int32),
            pltpu.VMEM((16,), jnp.int32),
            pltpu.VMEM((16, 16), jnp.int32),
            pltpu.VMEM((_TPW, _TILE), jnp.int32),
            pltpu.VMEM((_TPW, _TILE), jnp.int32),
            pltpu.VMEM((_SPW, _FEAT), jnp.float32),
            pltpu.VMEM((_SPW, 16), jnp.float32),
            pltpu.VMEM((16,), jnp.int32),
            pltpu.VMEM_SHARED((16, 16), jnp.int32),
            pltpu.SemaphoreType.DMA,
        ],
        compiler_params=sc_params,
    )

    normx = norm[:, 0] + 0.0
    normy = norm[:, 1] + 0.0
    viewp = jnp.pad(viewdir, ((0, 0), (0, 13)))
    perm, tid2d, xsf, xsv = route(normx, normy, feat, viewp)
    tile_eid = tid2d[:, :_TPW].reshape(_TILES)

    o0 = _IN1 * _HID
    w1 = kn_params[:, :o0].reshape(_NETS, _IN1, _HID)
    w1 = jnp.pad(w1, ((0, 0), (0, _IN1P - _IN1), (0, 0)))
    b1 = kn_params[:, o0:o0 + _HID].reshape(_NETS, 1, _HID)
    w2 = kn_params[:, o0 + _HID:].reshape(_NETS, _HID, 3)
    w2 = jnp.pad(w2, ((0, 0), (0, 0), (0, 13)))

    ys = pl.pallas_call(
        _mlp_body,
        grid_spec=pltpu.PrefetchScalarGridSpec(
            num_scalar_prefetch=1,
            grid=(_TILES,),
            in_specs=[
                pl.BlockSpec((_TILE, _FEAT), lambda t, eid: (t, 0)),
                pl.BlockSpec((_TILE, 16), lambda t, eid: (t, 0)),
                pl.BlockSpec((_NETS, _IN1P, _HID), lambda t, eid: (0, 0, 0)),
                pl.BlockSpec((_NETS, 1, _HID), lambda t, eid: (0, 0, 0)),
                pl.BlockSpec((_NETS, _HID, 16), lambda t, eid: (0, 0, 0)),
            ],
            out_specs=pl.BlockSpec((_TILE, 16), lambda t, eid: (t, 0)),
        ),
        out_shape=jax.ShapeDtypeStruct((_SLOTS, 16), jnp.float32),
    )(tile_eid, xsf, xsv, w1, b1, w2)

    scatter = pl.kernel(
        _scatter_body,
        out_type=jax.ShapeDtypeStruct((_N + 8, 16), jnp.float32),
        mesh=mesh,
        scratch_types=[
            pltpu.VMEM((_TPW, _TILE), jnp.int32),
            pltpu.VMEM((_SPW, 16), jnp.float32),
            pltpu.SemaphoreType.DMA,
        ],
        compiler_params=sc_params,
    )
    outp = scatter(perm, ys)
    return outp[:_N, :3]


# speedup vs baseline: 1.0099x; 1.0099x over previous
import functools

import jax
import jax.numpy as jnp
from jax import lax
from jax.experimental import pallas as pl
from jax.experimental.pallas import tpu as pltpu
from jax.experimental.pallas import tpu_sc as plsc

_N = 8192
_FEAT = 64
_HID = 64
_MAP = 8
_NETS = _MAP * _MAP
_FREQS = 4
_IN1 = _FEAT + 3 + 3 * 2 * _FREQS
_IN1P = 96
_TILE = 128
_HALF = 12288
_SLOTS = 2 * _HALF
_TILES = _SLOTS // _TILE
_TPW = 6
_SPW = _TPW * _TILE


def _route_body(nx_hbm, ny_hbm, feat_hbm, view_hbm,
                perm_hbm, tid_hbm, xsf_hbm, xsv_hbm,
                nx_v, ny_v, list0, list1, sent_v, rowbuf, cnt_rd,
                pc_v, pcc_v, featrows, viewrows, tid_row, cnt_sh, sem):
    c = lax.axis_index("c")
    s = lax.axis_index("s")
    half = c * _HALF
    lanes = lax.iota(jnp.int32, 16)
    eglob0 = 32 * c + 2 * s
    eglob1 = eglob0 + 1

    pltpu.sync_copy(nx_hbm, nx_v)
    pltpu.sync_copy(ny_hbm, ny_v)

    def step(i, carry):
        cur0, cur1 = carry
        vx = nx_v[pl.ds(i * 16, 16)]
        vy = ny_v[pl.ds(i * 16, 16)]
        cxi = jnp.clip(vx * float(_MAP), 0.0, _MAP - 1.0)
        cyi = jnp.clip(vy * float(_MAP), 0.0, _MAP - 1.0)
        eid = cxi.astype(jnp.int32) * _MAP + cyi.astype(jnp.int32)
        pid = i * 16 + lanes
        m0 = eid == eglob0
        mi0 = m0.astype(jnp.int32)
        r0 = plsc.cumsum(mi0) - mi0
        plsc.store_scatter(list0, [cur0 + r0], pid, mask=m0)
        cur0 = cur0 + plsc.all_reduce_population_count(m0)
        m1 = eid == eglob1
        mi1 = m1.astype(jnp.int32)
        r1 = plsc.cumsum(mi1) - mi1
        plsc.store_scatter(list1, [cur1 + r1], pid, mask=m1)
        cur1 = cur1 + plsc.all_reduce_population_count(m1)
        return cur0, cur1

    zero = jnp.zeros((16,), jnp.int32)
    cur0, cur1 = lax.fori_loop(0, _N // 16, step, (zero, zero))

    rowbuf[...] = jnp.where(lanes == 0, cur0, jnp.where(lanes == 1, cur1, 0))
    pltpu.sync_copy(rowbuf, cnt_sh.at[s])
    plsc.subcore_barrier()
    pltpu.sync_copy(cnt_sh, cnt_rd)

    k0 = lanes
    k1 = lanes + 16
    c0v = plsc.load_gather(cnt_rd, [k0 // 2, k0 % 2])
    c1v = plsc.load_gather(cnt_rd, [k1 // 2, k1 % 2])
    p0v = ((c0v + (_TILE - 1)) // _TILE) * _TILE
    p1v = ((c1v + (_TILE - 1)) // _TILE) * _TILE
    s0 = plsc.cumsum(p0v)
    off0 = s0 - p0v
    tot0 = jnp.max(s0)
    s1 = plsc.cumsum(p1v)
    off1 = s1 - p1v + tot0
    gt = tot0 + jnp.max(s1)

    def sel(vec0, vec1, k):
        v = jnp.where(k < 16, vec0, vec1)
        l = jnp.where(k < 16, k, k - 16)
        return jnp.sum(jnp.where(lanes == l, v, 0))

    dst0 = sel(off0, off1, 2 * s)
    dst1 = sel(off0, off1, 2 * s + 1)
    c0 = jnp.max(cur0)
    c1 = jnp.max(cur1)

    sent = jnp.full((16,), _N, jnp.int32)
    cb0 = pl.multiple_of((c0 // 16) * 16, 16)
    cb1 = pl.multiple_of((c1 // 16) * 16, 16)
    for m in range(9):
        pos0 = cb0 + 16 * m + lanes
        v0 = list0[pl.ds(cb0 + 16 * m, 16)]
        list0[pl.ds(cb0 + 16 * m, 16)] = jnp.where(pos0 >= c0, sent, v0)
        pos1 = cb1 + 16 * m + lanes
        v1 = list1[pl.ds(cb1 + 16 * m, 16)]
        list1[pl.ds(cb1 + 16 * m, 16)] = jnp.where(pos1 >= c1, sent, v1)
    for m in range(8):
        sent_v[pl.ds(16 * m, 16)] = sent

    nch0 = (c0 + (_TILE - 1)) // _TILE
    nch1 = (c1 + (_TILE - 1)) // _TILE
    t_half = c * (_HALF // _TILE)
    t0 = t_half + dst0 // _TILE
    t1 = t_half + dst1 // _TILE

    def copy0(j, carry):
        pltpu.sync_copy(list0.at[pl.ds(pl.multiple_of(j * _TILE, _TILE),
                                       _TILE)],
                        perm_hbm.at[t0 + j])
        return carry

    def copy1(j, carry):
        pltpu.sync_copy(list1.at[pl.ds(pl.multiple_of(j * _TILE, _TILE),
                                       _TILE)],
                        perm_hbm.at[t1 + j])
        return carry

    lax.fori_loop(0, nch0, copy0, 0)
    lax.fori_loop(0, nch1, copy1, 0)

    gtc = gt // _TILE

    def tail_fill(j, carry):
        pltpu.sync_copy(sent_v, perm_hbm.at[t_half + gtc + j * 16 + s])
        return carry

    ntail = (_HALF // _TILE - gtc + 15 - s) // 16
    lax.fori_loop(0, ntail, tail_fill, 0)

    plsc.subcore_barrier()

    tbase = (16 * c + s) * _TPW
    base = pl.multiple_of(_SPW * (16 * c + s), _TILE)
    pltpu.sync_copy(perm_hbm.at[pl.ds(tbase, _TPW)], pc_v)
    for m in range(_TPW):
        for q in range(_TILE // 16):
            v = pc_v[m, pl.ds(16 * q, 16)]
            pcc_v[m, pl.ds(16 * q, 16)] = jnp.minimum(v, _N - 1)

    copies = []
    for m in range(_TPW):
        copies.append(pltpu.async_copy(
            feat_hbm.at[pcc_v.at[m]],
            featrows.at[pl.ds(m * _TILE, _TILE)], sem))
        copies.append(pltpu.async_copy(
            view_hbm.at[pcc_v.at[m]],
            viewrows.at[pl.ds(m * _TILE, _TILE)], sem))
    for cp in copies:
        cp.wait()
    pltpu.sync_copy(featrows, xsf_hbm.at[pl.ds(base, _SPW)])
    pltpu.sync_copy(viewrows, xsv_hbm.at[pl.ds(base, _SPW)])

    trow = jnp.zeros((16,), jnp.int32)
    for m in range(_TPW):
        v = pcc_v[m, pl.ds(0, 16)]
        p = jnp.sum(jnp.where(lanes == 0, v, 0))
        pvec = jnp.zeros((16,), jnp.int32) + p
        gx = plsc.load_gather(nx_v, [pvec])
        gy = plsc.load_gather(ny_v, [pvec])
        ex = jnp.clip(gx * float(_MAP), 0.0, _MAP - 1.0)
        ey = jnp.clip(gy * float(_MAP), 0.0, _MAP - 1.0)
        em = ex.astype(jnp.int32) * _MAP + ey.astype(jnp.int32)
        trow = jnp.where(lanes == m, em, trow)
    tid_row[...] = trow
    pltpu.sync_copy(tid_row, tid_hbm.at[16 * c + s])


def _mlp_body(eid_ref, xf_ref, xv_ref, w1_ref, b1_ref, w2_ref, ys_ref):
    t = pl.program_id(0)
    e = eid_ref[t]
    xf = xf_ref[...]
    xv = xv_ref[...]
    v = xv[:, :3]
    ang = jnp.concatenate([v * ((2.0 ** k) * jnp.pi) for k in range(_FREQS)],
                          axis=1)
    x = jnp.concatenate(
        [xf, v, jnp.sin(ang), jnp.cos(ang),
         jnp.zeros((xf.shape[0], _IN1P - _IN1), xf.dtype)], axis=1)
    h = jax.lax.dot_general(x, w1_ref[e], (((1,), (0,)), ((), ())),
                            preferred_element_type=jnp.float32)
    h = jax.nn.relu(h + b1_ref[e])
    ys_ref[...] = jax.lax.dot_general(h, w2_ref[e], (((1,), (0,)), ((), ())),
                                      preferred_element_type=jnp.float32)


def _scatter_body(perm_hbm, ys_hbm, outp_hbm, pc_v, rows_v, sem):
    c = lax.axis_index("c")
    s = lax.axis_index("s")
    w = 16 * c + s
    base = pl.multiple_of(w * _SPW, _TILE)
    pltpu.sync_copy(perm_hbm.at[pl.ds(w * _TPW, _TPW)], pc_v)
    pltpu.sync_copy(ys_hbm.at[pl.ds(base, _SPW)], rows_v)
    copies = [pltpu.async_copy(rows_v.at[pl.ds(m * _TILE, _TILE)],
                               outp_hbm.at[pc_v.at[m]], sem)
              for m in range(_TPW)]
    for cp in copies:
        cp.wait()


def kernel(norm, feat, viewdir, kn_params):
    mesh = plsc.VectorSubcoreMesh(core_axis_name="c", subcore_axis_name="s")
    sc_params = pltpu.CompilerParams(needs_layout_passes=False,
                                     use_tc_tiling_on_sc=False)

    route = pl.kernel(
        _route_body,
        out_type=[
            jax.ShapeDtypeStruct((_TILES, _TILE), jnp.int32),
            jax.ShapeDtypeStruct((32, 16), jnp.int32),
            jax.ShapeDtypeStruct((_SLOTS, _FEAT), jnp.float32),
            jax.ShapeDtypeStruct((_SLOTS, 16), jnp.float32),
        ],
        mesh=mesh,
        scratch_types=[
            pltpu.VMEM((_N,), jnp.float32),
            pltpu.VMEM((_N,), jnp.float32),
            pltpu.VMEM((_N + _TILE,), jnp.int32),
            pltpu.VMEM((_N + _TILE,), jnp.int32),
            pltpu.VMEM((_TILE,), jnp.int32),
            pltpu.VMEM((16,), jnp.int32),
            pltpu.VMEM((16, 16), jnp.int32),
            pltpu.VMEM((_TPW, _TILE), jnp.int32),
            pltpu.VMEM((_TPW, _TILE), jnp.int32),
            pltpu.VMEM((_SPW, _FEAT), jnp.float32),
            pltpu.VMEM((_SPW, 16), jnp.float32),
            pltpu.VMEM((16,), jnp.int32),
            pltpu.VMEM_SHARED((16, 16), jnp.int32),
            pltpu.SemaphoreType.DMA,
        ],
        compiler_params=sc_params,
    )

    normx = norm[:, 0] + 0.0
    normy = norm[:, 1] + 0.0
    viewp = jnp.pad(viewdir, ((0, 0), (0, 13)))
    perm, tid2d, xsf, xsv = route(normx, normy, feat, viewp)
    tile_eid = tid2d[:, :_TPW].reshape(_TILES)

    o0 = _IN1 * _HID
    w1 = kn_params[:, :o0].reshape(_NETS, _IN1, _HID)
    w1 = jnp.pad(w1, ((0, 0), (0, _IN1P - _IN1), (0, 0)))
    b1 = kn_params[:, o0:o0 + _HID].reshape(_NETS, 1, _HID)
    w2 = kn_params[:, o0 + _HID:].reshape(_NETS, _HID, 3)
    w2 = jnp.pad(w2, ((0, 0), (0, 0), (0, 13)))

    ys = pl.pallas_call(
        _mlp_body,
        grid_spec=pltpu.PrefetchScalarGridSpec(
            num_scalar_prefetch=1,
            grid=(_TILES,),
            in_specs=[
                pl.BlockSpec((_TILE, _FEAT), lambda t, eid: (t, 0)),
                pl.BlockSpec((_TILE, 16), lambda t, eid: (t, 0)),
                pl.BlockSpec((_NETS, _IN1P, _HID), lambda t, eid: (0, 0, 0)),
                pl.BlockSpec((_NETS, 1, _HID), lambda t, eid: (0, 0, 0)),
                pl.BlockSpec((_NETS, _HID, 16), lambda t, eid: (0, 0, 0)),
            ],
            out_specs=pl.BlockSpec((_TILE, 16), lambda t, eid: (t, 0)),
        ),
        out_shape=jax.ShapeDtypeStruct((_SLOTS, 16), jnp.float32),
    )(tile_eid, xsf, xsv, w1, b1, w2)

    scatter = pl.kernel(
        _scatter_body,
        out_type=jax.ShapeDtypeStruct((_N + 8, 16), jnp.float32),
        mesh=mesh,
        scratch_types=[
            pltpu.VMEM((_TPW, _TILE), jnp.int32),
            pltpu.VMEM((_SPW, 16), jnp.float32),
            pltpu.SemaphoreType.DMA,
        ],
        compiler_params=sc_params,
    )
    outp = scatter(perm, ys)
    return outp[:_N, :3]


# speedup vs baseline: 1.0935x; 1.0828x over previous
import functools

import jax
import jax.numpy as jnp
from jax import lax
from jax.experimental import pallas as pl
from jax.experimental.pallas import tpu as pltpu
from jax.experimental.pallas import tpu_sc as plsc

_N = 8192
_FEAT = 64
_HID = 64
_MAP = 8
_NETS = _MAP * _MAP
_FREQS = 4
_IN1 = _FEAT + 3 + 3 * 2 * _FREQS
_IN1P = 96
_TILE = 128
_HALF = 12288
_SLOTS = 2 * _HALF
_TILES = _SLOTS // _TILE
_TPW = 6
_SPW = _TPW * _TILE


def _route_body(nx_hbm, ny_hbm, feat_hbm, view_hbm,
                perm_hbm, tid_hbm, xsf_hbm, xsv_hbm,
                nx_v, ny_v, list0, list1, sent_v, rowbuf, cnt_rd,
                pc_v, pcc_v, featrows, viewrows, tid_row, cnt_sh, sem):
    c = lax.axis_index("c")
    s = lax.axis_index("s")
    half = c * _HALF
    lanes = lax.iota(jnp.int32, 16)
    eglob0 = 32 * c + 2 * s
    eglob1 = eglob0 + 1

    pltpu.sync_copy(nx_hbm, nx_v)
    pltpu.sync_copy(ny_hbm, ny_v)

    def step(i, carry):
        cur0, cur1 = carry
        vx = nx_v[pl.ds(i * 16, 16)]
        vy = ny_v[pl.ds(i * 16, 16)]
        cxi = jnp.clip(vx * float(_MAP), 0.0, _MAP - 1.0)
        cyi = jnp.clip(vy * float(_MAP), 0.0, _MAP - 1.0)
        eid = cxi.astype(jnp.int32) * _MAP + cyi.astype(jnp.int32)
        pid = i * 16 + lanes
        m0 = eid == eglob0
        mi0 = m0.astype(jnp.int32)
        r0 = plsc.cumsum(mi0) - mi0
        plsc.store_scatter(list0, [cur0 + r0], pid, mask=m0)
        cur0 = cur0 + plsc.all_reduce_population_count(m0)
        m1 = eid == eglob1
        mi1 = m1.astype(jnp.int32)
        r1 = plsc.cumsum(mi1) - mi1
        plsc.store_scatter(list1, [cur1 + r1], pid, mask=m1)
        cur1 = cur1 + plsc.all_reduce_population_count(m1)
        return cur0, cur1

    zero = jnp.zeros((16,), jnp.int32)
    cur0, cur1 = lax.fori_loop(0, _N // 16, step, (zero, zero))

    rowbuf[...] = jnp.where(lanes == 0, cur0, jnp.where(lanes == 1, cur1, 0))
    pltpu.sync_copy(rowbuf, cnt_sh.at[s])
    plsc.subcore_barrier()
    pltpu.sync_copy(cnt_sh, cnt_rd)

    k0 = lanes
    k1 = lanes + 16
    c0v = plsc.load_gather(cnt_rd, [k0 // 2, k0 % 2])
    c1v = plsc.load_gather(cnt_rd, [k1 // 2, k1 % 2])
    p0v = ((c0v + (_TILE - 1)) // _TILE) * _TILE
    p1v = ((c1v + (_TILE - 1)) // _TILE) * _TILE
    s0 = plsc.cumsum(p0v)
    off0 = s0 - p0v
    tot0 = jnp.max(s0)
    s1 = plsc.cumsum(p1v)
    off1 = s1 - p1v + tot0
    gt = tot0 + jnp.max(s1)

    def sel(vec0, vec1, k):
        v = jnp.where(k < 16, vec0, vec1)
        l = jnp.where(k < 16, k, k - 16)
        return jnp.sum(jnp.where(lanes == l, v, 0))

    dst0 = sel(off0, off1, 2 * s)
    dst1 = sel(off0, off1, 2 * s + 1)
    c0 = jnp.max(cur0)
    c1 = jnp.max(cur1)

    sent = jnp.full((16,), _N, jnp.int32)
    cb0 = pl.multiple_of((c0 // 16) * 16, 16)
    cb1 = pl.multiple_of((c1 // 16) * 16, 16)
    for m in range(9):
        pos0 = cb0 + 16 * m + lanes
        v0 = list0[pl.ds(cb0 + 16 * m, 16)]
        list0[pl.ds(cb0 + 16 * m, 16)] = jnp.where(pos0 >= c0, sent, v0)
        pos1 = cb1 + 16 * m + lanes
        v1 = list1[pl.ds(cb1 + 16 * m, 16)]
        list1[pl.ds(cb1 + 16 * m, 16)] = jnp.where(pos1 >= c1, sent, v1)
    for m in range(8):
        sent_v[pl.ds(16 * m, 16)] = sent

    nch0 = (c0 + (_TILE - 1)) // _TILE
    nch1 = (c1 + (_TILE - 1)) // _TILE
    t_half = c * (_HALF // _TILE)
    t0 = t_half + dst0 // _TILE
    t1 = t_half + dst1 // _TILE

    def copy0(j, carry):
        pltpu.sync_copy(list0.at[pl.ds(pl.multiple_of(j * _TILE, _TILE),
                                       _TILE)],
                        perm_hbm.at[t0 + j])
        return carry

    def copy1(j, carry):
        pltpu.sync_copy(list1.at[pl.ds(pl.multiple_of(j * _TILE, _TILE),
                                       _TILE)],
                        perm_hbm.at[t1 + j])
        return carry

    lax.fori_loop(0, nch0, copy0, 0)
    lax.fori_loop(0, nch1, copy1, 0)

    gtc = gt // _TILE

    def tail_fill(j, carry):
        pltpu.sync_copy(sent_v, perm_hbm.at[t_half + gtc + j * 16 + s])
        return carry

    ntail = (_HALF // _TILE - gtc + 15 - s) // 16
    lax.fori_loop(0, ntail, tail_fill, 0)

    plsc.subcore_barrier()

    tbase = (16 * c + s) * _TPW
    base = pl.multiple_of(_SPW * (16 * c + s), _TILE)
    pltpu.sync_copy(perm_hbm.at[pl.ds(tbase, _TPW)], pc_v)
    for m in range(_TPW):
        for q in range(_TILE // 16):
            v = pc_v[m, pl.ds(16 * q, 16)]
            pcc_v[m, pl.ds(16 * q, 16)] = jnp.minimum(v, _N - 1)

    copies = []
    for m in range(_TPW):
        copies.append(pltpu.async_copy(
            feat_hbm.at[pcc_v.at[m]],
            featrows.at[pl.ds(m * _TILE, _TILE)], sem))
        copies.append(pltpu.async_copy(
            view_hbm.at[pcc_v.at[m]],
            viewrows.at[pl.ds(m * _TILE, _TILE)], sem))
    for cp in copies:
        cp.wait()
    pltpu.sync_copy(featrows, xsf_hbm.at[pl.ds(base, _SPW)])
    pltpu.sync_copy(viewrows, xsv_hbm.at[pl.ds(base, _SPW)])

    trow = jnp.zeros((16,), jnp.int32)
    for m in range(_TPW):
        v = pcc_v[m, pl.ds(0, 16)]
        p = jnp.sum(jnp.where(lanes == 0, v, 0))
        pvec = jnp.zeros((16,), jnp.int32) + p
        gx = plsc.load_gather(nx_v, [pvec])
        gy = plsc.load_gather(ny_v, [pvec])
        ex = jnp.clip(gx * float(_MAP), 0.0, _MAP - 1.0)
        ey = jnp.clip(gy * float(_MAP), 0.0, _MAP - 1.0)
        em = ex.astype(jnp.int32) * _MAP + ey.astype(jnp.int32)
        trow = jnp.where(lanes == m, em, trow)
    tid_row[...] = trow
    pltpu.sync_copy(tid_row, tid_hbm.at[16 * c + s])


def _mlp_body(eid_ref, xf_ref, xv_ref, w1_ref, b1_ref, w2_ref, ys_ref):
    t = pl.program_id(0)
    e = eid_ref[t]
    xf = xf_ref[...]
    xv = xv_ref[...]
    v = xv[:, :3]
    ang = jnp.concatenate([v * ((2.0 ** k) * jnp.pi) for k in range(_FREQS)],
                          axis=1)
    x = jnp.concatenate(
        [xf, v, jnp.sin(ang), jnp.cos(ang),
         jnp.zeros((xf.shape[0], _IN1P - _IN1), xf.dtype)], axis=1)
    h = jax.lax.dot_general(x, w1_ref[e], (((1,), (0,)), ((), ())),
                            preferred_element_type=jnp.float32)
    h = jax.nn.relu(h + b1_ref[e])
    ys_ref[...] = jax.lax.dot_general(h, w2_ref[e], (((1,), (0,)), ((), ())),
                                      preferred_element_type=jnp.float32)


def _scatter_body(perm_hbm, ys_hbm, outp_hbm, pc_v, pcd_v, rows_v, sem):
    c = lax.axis_index("c")
    s = lax.axis_index("s")
    w = 16 * c + s
    base = pl.multiple_of(w * _SPW, _TILE)
    pltpu.sync_copy(perm_hbm.at[pl.ds(w * _TPW, _TPW)], pc_v)
    pltpu.sync_copy(ys_hbm.at[pl.ds(base, _SPW)], rows_v)
    for m in range(_TPW):
        for q in range(_TILE // 16):
            v = pc_v[m, pl.ds(16 * q, 16)]
            pcd_v[m, pl.ds(16 * q, 16)] = jnp.where(v >= _N, _N + w, v)
    copies = [pltpu.async_copy(rows_v.at[pl.ds(m * _TILE, _TILE)],
                               outp_hbm.at[pcd_v.at[m]], sem)
              for m in range(_TPW)]
    for cp in copies:
        cp.wait()


def kernel(norm, feat, viewdir, kn_params):
    mesh = plsc.VectorSubcoreMesh(core_axis_name="c", subcore_axis_name="s")
    sc_params = pltpu.CompilerParams(needs_layout_passes=False,
                                     use_tc_tiling_on_sc=False)

    route = pl.kernel(
        _route_body,
        out_type=[
            jax.ShapeDtypeStruct((_TILES, _TILE), jnp.int32),
            jax.ShapeDtypeStruct((32, 16), jnp.int32),
            jax.ShapeDtypeStruct((_SLOTS, _FEAT), jnp.float32),
            jax.ShapeDtypeStruct((_SLOTS, 16), jnp.float32),
        ],
        mesh=mesh,
        scratch_types=[
            pltpu.VMEM((_N,), jnp.float32),
            pltpu.VMEM((_N,), jnp.float32),
            pltpu.VMEM((_N + _TILE,), jnp.int32),
            pltpu.VMEM((_N + _TILE,), jnp.int32),
            pltpu.VMEM((_TILE,), jnp.int32),
            pltpu.VMEM((16,), jnp.int32),
            pltpu.VMEM((16, 16), jnp.int32),
            pltpu.VMEM((_TPW, _TILE), jnp.int32),
            pltpu.VMEM((_TPW, _TILE), jnp.int32),
            pltpu.VMEM((_SPW, _FEAT), jnp.float32),
            pltpu.VMEM((_SPW, 16), jnp.float32),
            pltpu.VMEM((16,), jnp.int32),
            pltpu.VMEM_SHARED((16, 16), jnp.int32),
            pltpu.SemaphoreType.DMA,
        ],
        compiler_params=sc_params,
    )

    normx = norm[:, 0] + 0.0
    normy = norm[:, 1] + 0.0
    viewp = jnp.pad(viewdir, ((0, 0), (0, 13)))
    perm, tid2d, xsf, xsv = route(normx, normy, feat, viewp)
    tile_eid = tid2d[:, :_TPW].reshape(_TILES)

    o0 = _IN1 * _HID
    w1 = kn_params[:, :o0].reshape(_NETS, _IN1, _HID)
    w1 = jnp.pad(w1, ((0, 0), (0, _IN1P - _IN1), (0, 0)))
    b1 = kn_params[:, o0:o0 + _HID].reshape(_NETS, 1, _HID)
    w2 = kn_params[:, o0 + _HID:].reshape(_NETS, _HID, 3)
    w2 = jnp.pad(w2, ((0, 0), (0, 0), (0, 13)))

    ys = pl.pallas_call(
        _mlp_body,
        grid_spec=pltpu.PrefetchScalarGridSpec(
            num_scalar_prefetch=1,
            grid=(_TILES,),
            in_specs=[
                pl.BlockSpec((_TILE, _FEAT), lambda t, eid: (t, 0)),
                pl.BlockSpec((_TILE, 16), lambda t, eid: (t, 0)),
                pl.BlockSpec((_NETS, _IN1P, _HID), lambda t, eid: (0, 0, 0)),
                pl.BlockSpec((_NETS, 1, _HID), lambda t, eid: (0, 0, 0)),
                pl.BlockSpec((_NETS, _HID, 16), lambda t, eid: (0, 0, 0)),
            ],
            out_specs=pl.BlockSpec((_TILE, 16), lambda t, eid: (t, 0)),
        ),
        out_shape=jax.ShapeDtypeStruct((_SLOTS, 16), jnp.float32),
    )(tile_eid, xsf, xsv, w1, b1, w2)

    scatter = pl.kernel(
        _scatter_body,
        out_type=jax.ShapeDtypeStruct((_N + 32, 16), jnp.float32),
        mesh=mesh,
        scratch_types=[
            pltpu.VMEM((_TPW, _TILE), jnp.int32),
            pltpu.VMEM((_TPW, _TILE), jnp.int32),
            pltpu.VMEM((_SPW, 16), jnp.float32),
            pltpu.SemaphoreType.DMA,
        ],
        compiler_params=sc_params,
    )
    outp = scatter(perm, ys)
    return outp[:_N, :3]


# speedup vs baseline: 2.3577x; 2.1562x over previous
import functools

import jax
import jax.numpy as jnp
from jax import lax
from jax.experimental import pallas as pl
from jax.experimental.pallas import tpu as pltpu
from jax.experimental.pallas import tpu_sc as plsc

_N = 8192
_FEAT = 64
_HID = 64
_MAP = 8
_NETS = _MAP * _MAP
_FREQS = 4
_IN1 = _FEAT + 3 + 3 * 2 * _FREQS
_IN1P = 96
_TILE = 128
_HALF = 12288
_SLOTS = 2 * _HALF
_TILES = _SLOTS // _TILE
_TPW = 6
_SPW = _TPW * _TILE


def _route_body(nx_hbm, ny_hbm, feat_hbm, view_hbm,
                perm_hbm, tid_hbm, xsf_hbm, xsv_hbm,
                nx_v, ny_v, list0, list1, sent_v, rowbuf, cnt_rd,
                pc_v, pcc_v, featrows, viewrows, tid_row, cnt_sh, sem):
    c = lax.axis_index("c")
    s = lax.axis_index("s")
    half = c * _HALF
    lanes = lax.iota(jnp.int32, 16)
    eglob0 = 32 * c + 2 * s
    eglob1 = eglob0 + 1

    pltpu.sync_copy(nx_hbm, nx_v)
    pltpu.sync_copy(ny_hbm, ny_v)

    def step(i, carry):
        cur0, cur1 = carry
        vx = nx_v[pl.ds(i * 16, 16)]
        vy = ny_v[pl.ds(i * 16, 16)]
        cxi = jnp.clip(vx * float(_MAP), 0.0, _MAP - 1.0)
        cyi = jnp.clip(vy * float(_MAP), 0.0, _MAP - 1.0)
        eid = cxi.astype(jnp.int32) * _MAP + cyi.astype(jnp.int32)
        pid = i * 16 + lanes
        m0 = eid == eglob0
        mi0 = m0.astype(jnp.int32)
        r0 = plsc.cumsum(mi0) - mi0
        plsc.store_scatter(list0, [cur0 + r0], pid, mask=m0)
        cur0 = cur0 + plsc.all_reduce_population_count(m0)
        m1 = eid == eglob1
        mi1 = m1.astype(jnp.int32)
        r1 = plsc.cumsum(mi1) - mi1
        plsc.store_scatter(list1, [cur1 + r1], pid, mask=m1)
        cur1 = cur1 + plsc.all_reduce_population_count(m1)
        return cur0, cur1

    zero = jnp.zeros((16,), jnp.int32)
    cur0, cur1 = lax.fori_loop(0, _N // 16, step, (zero, zero))

    rowbuf[...] = jnp.where(lanes == 0, cur0, jnp.where(lanes == 1, cur1, 0))
    pltpu.sync_copy(rowbuf, cnt_sh.at[s])
    plsc.subcore_barrier()
    pltpu.sync_copy(cnt_sh, cnt_rd)

    k0 = lanes
    k1 = lanes + 16
    c0v = plsc.load_gather(cnt_rd, [k0 // 2, k0 % 2])
    c1v = plsc.load_gather(cnt_rd, [k1 // 2, k1 % 2])
    p0v = ((c0v + (_TILE - 1)) // _TILE) * _TILE
    p1v = ((c1v + (_TILE - 1)) // _TILE) * _TILE
    s0 = plsc.cumsum(p0v)
    off0 = s0 - p0v
    tot0 = jnp.max(s0)
    s1 = plsc.cumsum(p1v)
    off1 = s1 - p1v + tot0
    gt = tot0 + jnp.max(s1)

    def sel(vec0, vec1, k):
        v = jnp.where(k < 16, vec0, vec1)
        l = jnp.where(k < 16, k, k - 16)
        return jnp.sum(jnp.where(lanes == l, v, 0))

    dst0 = sel(off0, off1, 2 * s)
    dst1 = sel(off0, off1, 2 * s + 1)
    c0 = jnp.max(cur0)
    c1 = jnp.max(cur1)

    sent = jnp.full((16,), _N, jnp.int32)
    cb0 = pl.multiple_of((c0 // 16) * 16, 16)
    cb1 = pl.multiple_of((c1 // 16) * 16, 16)
    for m in range(9):
        pos0 = cb0 + 16 * m + lanes
        v0 = list0[pl.ds(cb0 + 16 * m, 16)]
        list0[pl.ds(cb0 + 16 * m, 16)] = jnp.where(pos0 >= c0, sent, v0)
        pos1 = cb1 + 16 * m + lanes
        v1 = list1[pl.ds(cb1 + 16 * m, 16)]
        list1[pl.ds(cb1 + 16 * m, 16)] = jnp.where(pos1 >= c1, sent, v1)
    for m in range(8):
        sent_v[pl.ds(16 * m, 16)] = sent

    nch0 = (c0 + (_TILE - 1)) // _TILE
    nch1 = (c1 + (_TILE - 1)) // _TILE
    t_half = c * (_HALF // _TILE)
    t0 = t_half + dst0 // _TILE
    t1 = t_half + dst1 // _TILE

    def copy0(j, carry):
        pltpu.sync_copy(list0.at[pl.ds(pl.multiple_of(j * _TILE, _TILE),
                                       _TILE)],
                        perm_hbm.at[t0 + j])
        return carry

    def copy1(j, carry):
        pltpu.sync_copy(list1.at[pl.ds(pl.multiple_of(j * _TILE, _TILE),
                                       _TILE)],
                        perm_hbm.at[t1 + j])
        return carry

    lax.fori_loop(0, nch0, copy0, 0)
    lax.fori_loop(0, nch1, copy1, 0)

    gtc = gt // _TILE

    def tail_fill(j, carry):
        pltpu.sync_copy(sent_v, perm_hbm.at[t_half + gtc + j * 16 + s])
        return carry

    ntail = (_HALF // _TILE - gtc + 15 - s) // 16
    lax.fori_loop(0, ntail, tail_fill, 0)

    plsc.subcore_barrier()

    tbase = (16 * c + s) * _TPW
    base = pl.multiple_of(_SPW * (16 * c + s), _TILE)
    pltpu.sync_copy(perm_hbm.at[pl.ds(tbase, _TPW)], pc_v)
    for m in range(_TPW):
        for q in range(_TILE // 16):
            v = pc_v[m, pl.ds(16 * q, 16)]
            spread = (base + m * _TILE + q * 16 + lanes) & (_N - 1)
            pcc_v[m, pl.ds(16 * q, 16)] = jnp.where(v >= _N, spread, v)

    copies = []
    for m in range(_TPW):
        copies.append(pltpu.async_copy(
            feat_hbm.at[pcc_v.at[m]],
            featrows.at[pl.ds(m * _TILE, _TILE)], sem))
        copies.append(pltpu.async_copy(
            view_hbm.at[pcc_v.at[m]],
            viewrows.at[pl.ds(m * _TILE, _TILE)], sem))
    for cp in copies:
        cp.wait()
    pltpu.sync_copy(featrows, xsf_hbm.at[pl.ds(base, _SPW)])
    pltpu.sync_copy(viewrows, xsv_hbm.at[pl.ds(base, _SPW)])

    trow = jnp.zeros((16,), jnp.int32)
    for m in range(_TPW):
        v = pcc_v[m, pl.ds(0, 16)]
        p = jnp.sum(jnp.where(lanes == 0, v, 0))
        pvec = jnp.zeros((16,), jnp.int32) + p
        gx = plsc.load_gather(nx_v, [pvec])
        gy = plsc.load_gather(ny_v, [pvec])
        ex = jnp.clip(gx * float(_MAP), 0.0, _MAP - 1.0)
        ey = jnp.clip(gy * float(_MAP), 0.0, _MAP - 1.0)
        em = ex.astype(jnp.int32) * _MAP + ey.astype(jnp.int32)
        trow = jnp.where(lanes == m, em, trow)
    tid_row[...] = trow
    pltpu.sync_copy(tid_row, tid_hbm.at[16 * c + s])


def _mlp_body(eid_ref, xf_ref, xv_ref, w1_ref, b1_ref, w2_ref, ys_ref):
    t = pl.program_id(0)
    e = eid_ref[t]
    xf = xf_ref[...]
    xv = xv_ref[...]
    v = xv[:, :3]
    ang = jnp.concatenate([v * ((2.0 ** k) * jnp.pi) for k in range(_FREQS)],
                          axis=1)
    x = jnp.concatenate(
        [xf, v, jnp.sin(ang), jnp.cos(ang),
         jnp.zeros((xf.shape[0], _IN1P - _IN1), xf.dtype)], axis=1)
    h = jax.lax.dot_general(x, w1_ref[e], (((1,), (0,)), ((), ())),
                            preferred_element_type=jnp.float32)
    h = jax.nn.relu(h + b1_ref[e])
    ys_ref[...] = jax.lax.dot_general(h, w2_ref[e], (((1,), (0,)), ((), ())),
                                      preferred_element_type=jnp.float32)


def _scatter_body(perm_hbm, ys_hbm, outp_hbm, pc_v, pcd_v, rows_v, sem):
    c = lax.axis_index("c")
    s = lax.axis_index("s")
    w = 16 * c + s
    base = pl.multiple_of(w * _SPW, _TILE)
    pltpu.sync_copy(perm_hbm.at[pl.ds(w * _TPW, _TPW)], pc_v)
    pltpu.sync_copy(ys_hbm.at[pl.ds(base, _SPW)], rows_v)
    for m in range(_TPW):
        for q in range(_TILE // 16):
            v = pc_v[m, pl.ds(16 * q, 16)]
            pcd_v[m, pl.ds(16 * q, 16)] = jnp.where(v >= _N, _N + w, v)
    copies = [pltpu.async_copy(rows_v.at[pl.ds(m * _TILE, _TILE)],
                               outp_hbm.at[pcd_v.at[m]], sem)
              for m in range(_TPW)]
    for cp in copies:
        cp.wait()


def kernel(norm, feat, viewdir, kn_params):
    mesh = plsc.VectorSubcoreMesh(core_axis_name="c", subcore_axis_name="s")
    sc_params = pltpu.CompilerParams(needs_layout_passes=False,
                                     use_tc_tiling_on_sc=False)

    route = pl.kernel(
        _route_body,
        out_type=[
            jax.ShapeDtypeStruct((_TILES, _TILE), jnp.int32),
            jax.ShapeDtypeStruct((32, 16), jnp.int32),
            jax.ShapeDtypeStruct((_SLOTS, _FEAT), jnp.float32),
            jax.ShapeDtypeStruct((_SLOTS, 16), jnp.float32),
        ],
        mesh=mesh,
        scratch_types=[
            pltpu.VMEM((_N,), jnp.float32),
            pltpu.VMEM((_N,), jnp.float32),
            pltpu.VMEM((_N + _TILE,), jnp.int32),
            pltpu.VMEM((_N + _TILE,), jnp.int32),
            pltpu.VMEM((_TILE,), jnp.int32),
            pltpu.VMEM((16,), jnp.int32),
            pltpu.VMEM((16, 16), jnp.int32),
            pltpu.VMEM((_TPW, _TILE), jnp.int32),
            pltpu.VMEM((_TPW, _TILE), jnp.int32),
            pltpu.VMEM((_SPW, _FEAT), jnp.float32),
            pltpu.VMEM((_SPW, 16), jnp.float32),
            pltpu.VMEM((16,), jnp.int32),
            pltpu.VMEM_SHARED((16, 16), jnp.int32),
            pltpu.SemaphoreType.DMA,
        ],
        compiler_params=sc_params,
    )

    normx = norm[:, 0] + 0.0
    normy = norm[:, 1] + 0.0
    viewp = jnp.pad(viewdir, ((0, 0), (0, 13)))
    perm, tid2d, xsf, xsv = route(normx, normy, feat, viewp)
    tile_eid = tid2d[:, :_TPW].reshape(_TILES)

    o0 = _IN1 * _HID
    w1 = kn_params[:, :o0].reshape(_NETS, _IN1, _HID)
    w1 = jnp.pad(w1, ((0, 0), (0, _IN1P - _IN1), (0, 0)))
    b1 = kn_params[:, o0:o0 + _HID].reshape(_NETS, 1, _HID)
    w2 = kn_params[:, o0 + _HID:].reshape(_NETS, _HID, 3)
    w2 = jnp.pad(w2, ((0, 0), (0, 0), (0, 13)))

    ys = pl.pallas_call(
        _mlp_body,
        grid_spec=pltpu.PrefetchScalarGridSpec(
            num_scalar_prefetch=1,
            grid=(_TILES,),
            in_specs=[
                pl.BlockSpec((_TILE, _FEAT), lambda t, eid: (t, 0)),
                pl.BlockSpec((_TILE, 16), lambda t, eid: (t, 0)),
                pl.BlockSpec((_NETS, _IN1P, _HID), lambda t, eid: (0, 0, 0)),
                pl.BlockSpec((_NETS, 1, _HID), lambda t, eid: (0, 0, 0)),
                pl.BlockSpec((_NETS, _HID, 16), lambda t, eid: (0, 0, 0)),
            ],
            out_specs=pl.BlockSpec((_TILE, 16), lambda t, eid: (t, 0)),
        ),
        out_shape=jax.ShapeDtypeStruct((_SLOTS, 16), jnp.float32),
    )(tile_eid, xsf, xsv, w1, b1, w2)

    scatter = pl.kernel(
        _scatter_body,
        out_type=jax.ShapeDtypeStruct((_N + 32, 16), jnp.float32),
        mesh=mesh,
        scratch_types=[
            pltpu.VMEM((_TPW, _TILE), jnp.int32),
            pltpu.VMEM((_TPW, _TILE), jnp.int32),
            pltpu.VMEM((_SPW, 16), jnp.float32),
            pltpu.SemaphoreType.DMA,
        ],
        compiler_params=sc_params,
    )
    outp = scatter(perm, ys)
    return outp[:_N, :3]
